# baseline (speedup 1.0000x reference)
import functools

import jax
import jax.numpy as jnp
from jax import lax
from jax.experimental import pallas as pl
from jax.experimental.pallas import tpu as pltpu
from jax.experimental.pallas import tpu_sc as plsc

_NUM_CORES = 2
_NUM_SUBCORES = 16
_NW = _NUM_CORES * _NUM_SUBCORES
_CHUNK = 128


@functools.lru_cache(maxsize=None)
def _make_gather(total: int, emb_dim: int):
    b_per_w = total // _NW
    n_chunks = b_per_w // _CHUNK
    mesh = plsc.VectorSubcoreMesh(core_axis_name="c", subcore_axis_name="s")

    @functools.partial(
        pl.kernel,
        mesh=mesh,
        out_type=jax.ShapeDtypeStruct((total, emb_dim), jnp.float32),
        scratch_types=[
            pltpu.VMEM((n_chunks, _CHUNK), jnp.int32),
            pltpu.VMEM((b_per_w, emb_dim), jnp.float32),
            pltpu.SemaphoreType.DMA,
        ],
        compiler_params=pltpu.CompilerParams(use_tc_tiling_on_sc=False),
    )
    def gather_kernel(table_hbm, idx_hbm, out_hbm, idx_v, rows_v, sem):
        wid = lax.axis_index("s") * _NUM_CORES + lax.axis_index("c")
        base = wid * b_per_w
        pltpu.sync_copy(idx_hbm.at[wid], idx_v)
        copies = []
        for j in range(n_chunks):
            copies.append(
                pltpu.async_copy(
                    table_hbm.at[idx_v.at[j]],
                    rows_v.at[pl.ds(j * _CHUNK, _CHUNK)],
                    sem,
                )
            )
        for c in copies:
            c.wait()
        pltpu.sync_copy(rows_v, out_hbm.at[pl.ds(base, b_per_w)])

    return gather_kernel


def kernel(inp, emb):
    batch, seq = inp.shape
    total = batch * seq
    emb_dim = emb.shape[1]
    idx = inp.reshape(_NW, total // _NW // _CHUNK, _CHUNK).astype(jnp.int32)
    out = _make_gather(total, emb_dim)(emb, idx)
    return out.reshape(batch, seq, emb_dim)


# speedup vs baseline: 1.0286x; 1.0286x over previous
import functools

import jax
import jax.numpy as jnp
from jax import lax
from jax.experimental import pallas as pl
from jax.experimental.pallas import tpu as pltpu
from jax.experimental.pallas import tpu_sc as plsc

_NUM_CORES = 2
_NUM_SUBCORES = 16
_NW = _NUM_CORES * _NUM_SUBCORES
_L = 16


@functools.lru_cache(maxsize=None)
def _make_gather(total: int, emb_dim: int):
    b_per_w = total // _NW
    n_groups = b_per_w // _L
    mesh = plsc.VectorSubcoreMesh(core_axis_name="c", subcore_axis_name="s")

    @functools.partial(
        pl.kernel,
        mesh=mesh,
        out_type=jax.ShapeDtypeStruct((total, emb_dim), jnp.float32),
        scratch_types=[
            pltpu.VMEM((b_per_w,), jnp.int32),
            pltpu.SemaphoreType.DMA,
        ],
    )
    def gather_kernel(table_hbm, idx_hbm, out_hbm, idx_v, sem):
        wid = lax.axis_index("s") * _NUM_CORES + lax.axis_index("c")
        base = wid * b_per_w
        pltpu.sync_copy(idx_hbm.at[pl.ds(base, b_per_w)], idx_v)

        def fire(g, _):
            v = idx_v[pl.ds(g * _L, _L)]
            for l in range(_L):
                pltpu.async_copy(
                    table_hbm.at[v[l]], out_hbm.at[base + g * _L + l], sem
                )
            return 0

        lax.fori_loop(0, n_groups, fire, 0)

        def drain(i, _):
            pltpu.make_async_copy(
                table_hbm.at[0], out_hbm.at[base], sem
            ).wait()
            return 0

        lax.fori_loop(0, b_per_w, drain, 0)

    return gather_kernel


def kernel(inp, emb):
    batch, seq = inp.shape
    total = batch * seq
    emb_dim = emb.shape[1]
    idx = inp.reshape(total).astype(jnp.int32)
    out = _make_gather(total, emb_dim)(emb, idx)
    return out.reshape(batch, seq, emb_dim)


# speedup vs baseline: 1.0292x; 1.0005x over previous
import functools

import jax
import jax.numpy as jnp
from jax import lax
from jax.experimental import pallas as pl
from jax.experimental.pallas import tpu as pltpu
from jax.experimental.pallas import tpu_sc as plsc

_NUM_CORES = 2
_NUM_SUBCORES = 16
_NW = _NUM_CORES * _NUM_SUBCORES
_L = 16


@functools.lru_cache(maxsize=None)
def _make_gather(total: int, emb_dim: int):
    b_per_w = total // _NW
    n_groups = b_per_w // _L
    mesh = plsc.VectorSubcoreMesh(core_axis_name="c", subcore_axis_name="s")

    @functools.partial(
        pl.kernel,
        mesh=mesh,
        out_type=jax.ShapeDtypeStruct((total, emb_dim), jnp.float32),
        scratch_types=[
            pltpu.VMEM((b_per_w,), jnp.int32),
            pltpu.SemaphoreType.DMA,
        ],
    )
    def gather_kernel(table_hbm, idx_hbm, out_hbm, idx_v, sem):
        wid = lax.axis_index("s") * _NUM_CORES + lax.axis_index("c")
        base = wid * b_per_w
        pltpu.sync_copy(idx_hbm.at[pl.ds(base, b_per_w)], idx_v)

        def fire(g, _):
            v = idx_v[pl.ds(g * _L, _L)]
            for l in range(_L):
                pltpu.async_copy(
                    table_hbm.at[v[l]], out_hbm.at[base + g * _L + l], sem
                )
            return 0

        lax.fori_loop(0, n_groups, fire, 0)

        pltpu.make_async_copy(
            table_hbm.at[pl.ds(0, b_per_w)],
            out_hbm.at[pl.ds(base, b_per_w)],
            sem,
        ).wait()

    return gather_kernel


def kernel(inp, emb):
    batch, seq = inp.shape
    total = batch * seq
    emb_dim = emb.shape[1]
    idx = inp.reshape(total).astype(jnp.int32)
    out = _make_gather(total, emb_dim)(emb, idx)
    return out.reshape(batch, seq, emb_dim)


# speedup vs baseline: 1.7181x; 1.6695x over previous
import functools

import jax
import jax.numpy as jnp
from jax import lax
from jax.experimental import pallas as pl
from jax.experimental.pallas import tpu as pltpu
from jax.experimental.pallas import tpu_sc as plsc

_NUM_CORES = 2
_NUM_SUBCORES = 16
_NW = _NUM_CORES * _NUM_SUBCORES
_L = 16


@functools.lru_cache(maxsize=None)
def _make_gather(total: int, emb_dim: int):
    b_per_w = total // _NW
    n_groups = b_per_w // _L
    mesh = plsc.VectorSubcoreMesh(core_axis_name="c", subcore_axis_name="s")

    @functools.partial(
        pl.kernel,
        mesh=mesh,
        out_type=jax.ShapeDtypeStruct((total, emb_dim), jnp.float32),
        scratch_types=[
            pltpu.VMEM((b_per_w,), jnp.int32),
            pltpu.VMEM((b_per_w, emb_dim), jnp.float32),
            pltpu.SemaphoreType.DMA,
        ],
    )
    def gather_kernel(table_hbm, idx_hbm, out_hbm, idx_v, rows_v, sem):
        wid = lax.axis_index("s") * _NUM_CORES + lax.axis_index("c")
        base = wid * b_per_w
        pltpu.sync_copy(idx_hbm.at[pl.ds(base, b_per_w)], idx_v)

        def fire(g, _):
            v = idx_v[pl.ds(g * _L, _L)]
            for l in range(_L):
                pltpu.async_copy(
                    table_hbm.at[v[l]], rows_v.at[g * _L + l], sem
                )
            return 0

        lax.fori_loop(0, n_groups, fire, 0)

        pltpu.make_async_copy(
            table_hbm.at[pl.ds(0, b_per_w)], rows_v, sem
        ).wait()

        pltpu.sync_copy(rows_v, out_hbm.at[pl.ds(base, b_per_w)])

    return gather_kernel


def kernel(inp, emb):
    batch, seq = inp.shape
    total = batch * seq
    emb_dim = emb.shape[1]
    idx = inp.reshape(total).astype(jnp.int32)
    out = _make_gather(total, emb_dim)(emb, idx)
    return out.reshape(batch, seq, emb_dim)


# speedup vs baseline: 2.5912x; 1.5082x over previous
import functools

import jax
import jax.numpy as jnp
from jax import lax
from jax.experimental import pallas as pl
from jax.experimental.pallas import tpu as pltpu
from jax.experimental.pallas import tpu_sc as plsc

_NUM_CORES = 2
_NUM_SUBCORES = 16
_NW = _NUM_CORES * _NUM_SUBCORES
_L = 16


@functools.lru_cache(maxsize=None)
def _make_gather(total: int, emb_dim: int):
    b_per_w = total // _NW
    n_groups = b_per_w // _L
    mesh = plsc.VectorSubcoreMesh(core_axis_name="c", subcore_axis_name="s")

    @functools.partial(
        pl.kernel,
        mesh=mesh,
        out_type=jax.ShapeDtypeStruct((total, emb_dim), jnp.float32),
        scratch_types=[
            pltpu.VMEM((b_per_w,), jnp.int32),
            pltpu.VMEM((b_per_w, emb_dim), jnp.float32),
            [pltpu.SemaphoreType.DMA] * _L,
        ],
    )
    def gather_kernel(table_hbm, idx_hbm, out_hbm, idx_v, rows_v, sems):
        wid = lax.axis_index("s") * _NUM_CORES + lax.axis_index("c")
        base = wid * b_per_w
        pltpu.sync_copy(idx_hbm.at[pl.ds(base, b_per_w)], idx_v)

        def fire(g, _):
            v = idx_v[pl.ds(g * _L, _L)]
            for l in range(_L):
                pltpu.async_copy(
                    table_hbm.at[v[l]], rows_v.at[g * _L + l], sems[l]
                )
            return 0

        lax.fori_loop(0, n_groups, fire, 0)

        for l in range(_L):
            pltpu.make_async_copy(
                table_hbm.at[pl.ds(0, n_groups)],
                rows_v.at[pl.ds(0, n_groups)],
                sems[l],
            ).wait()

        pltpu.sync_copy(rows_v, out_hbm.at[pl.ds(base, b_per_w)])

    return gather_kernel


def kernel(inp, emb):
    batch, seq = inp.shape
    total = batch * seq
    emb_dim = emb.shape[1]
    idx = inp.reshape(total).astype(jnp.int32)
    out = _make_gather(total, emb_dim)(emb, idx)
    return out.reshape(batch, seq, emb_dim)
